# baseline (speedup 1.0000x reference)
import functools

import jax
import jax.numpy as jnp
from jax import lax
from jax.experimental import pallas as pl
from jax.experimental.pallas import tpu as pltpu
from jax.experimental.pallas import tpu_sc as plsc

D = 64
N = 4096 * 200
LPC = 128
NW = 32
CPW = N // (LPC * NW)
SCALE = 8.0
G = 3
O = 3
RU = 8

_mesh = plsc.VectorSubcoreMesh(core_axis_name="c", subcore_axis_name="s")


@functools.partial(
    pl.kernel,
    out_type=jax.ShapeDtypeStruct((N, D), jnp.float32),
    mesh=_mesh,
    compiler_params=pltpu.CompilerParams(use_tc_tiling_on_sc=False),
    scratch_types=[
        pltpu.VMEM((CPW, LPC), jnp.int32),
        pltpu.VMEM((G, LPC, D), jnp.float32),
        pltpu.VMEM((O, LPC, D), jnp.float32),
        pltpu.SemaphoreType.DMA((G,)),
        pltpu.SemaphoreType.DMA((O,)),
    ],
)
def _emb_lookup(x_hbm, table_hbm, out_hbm, idx_v, gbuf, obuf, gsem, osem):
    wid = lax.axis_index("s") * 2 + lax.axis_index("c")
    pltpu.sync_copy(x_hbm.at[pl.ds(wid * CPW, CPW)], idx_v)

    for j in range(G):
        pltpu.async_copy(table_hbm.at[idx_v.at[j]], gbuf.at[j], gsem.at[j])

    def it(i, carry):
        b = lax.rem(i, G)
        o = lax.rem(i, O)
        pltpu.make_async_copy(
            table_hbm.at[idx_v.at[i]], gbuf.at[b], gsem.at[b]).wait()

        @pl.when(i >= O)
        def _():
            pltpu.make_async_copy(
                obuf.at[o], out_hbm.at[pl.ds(0, LPC)], osem.at[o]).wait()

        def srow(r, c2):
            for rr in range(RU):
                row = r * RU + rr
                for cc in range(D // 16):
                    sl = pl.ds(cc * 16, 16)
                    obuf[o, row, sl] = gbuf[b, row, sl] * SCALE
            return c2

        lax.fori_loop(0, LPC // RU, srow, 0)

        base = (wid * CPW + i) * LPC
        pltpu.async_copy(obuf.at[o], out_hbm.at[pl.ds(base, LPC)], osem.at[o])

        @pl.when(i + G < CPW)
        def _():
            pltpu.async_copy(
                table_hbm.at[idx_v.at[i + G]], gbuf.at[b], gsem.at[b])

        return carry

    lax.fori_loop(0, CPW, it, 0)

    for j in range(O):
        pltpu.make_async_copy(
            obuf.at[j], out_hbm.at[pl.ds(0, LPC)], osem.at[j]).wait()


def kernel(x, table):
    x2 = x.reshape(N // LPC, LPC)
    out = _emb_lookup(x2, table)
    return out.reshape(4096, 200, D)
